# baseline (speedup 1.0000x reference)
import functools

import jax
import jax.numpy as jnp
from jax import lax
from jax.experimental import pallas as pl
from jax.experimental.pallas import tpu as pltpu
from jax.experimental.pallas import tpu_sc as plsc

_N = 128
_NCELL = 27
_TOPK = 32
_CUTOFF2 = 25.0
_INVALID_BASE = 100000.0
_PAD_KEY = 1e9
_NV = (_N * _NCELL) // 16
_CAND = _N * _NCELL


def _sqrt16(x):
    xi = lax.bitcast_convert_type(x, jnp.int32)
    yi = (xi >> 1) + jnp.int32(0x1FBD1DF5)
    y = lax.bitcast_convert_type(yi, jnp.float32)
    for _ in range(2):
        y = 0.5 * (y + x / y)
    return y


def _merge32(R0, V0, R1, V1, ck, cv):
    cs, cvs = plsc.sort_key_val(ck, cv)
    cr = lax.rev(cs, (0,))
    cvr = lax.rev(cvs, (0,))
    m1 = R1 <= cr
    lo_k = jnp.where(m1, R1, cr)
    lo_v = jnp.where(m1, V1, cvr)
    l1k, l1v = plsc.sort_key_val(lo_k, lo_v)
    l1kr = lax.rev(l1k, (0,))
    l1vr = lax.rev(l1v, (0,))
    m2 = R0 <= l1kr
    ak = jnp.where(m2, R0, l1kr)
    av = jnp.where(m2, V0, l1vr)
    bk = jnp.where(m2, l1kr, R0)
    bv = jnp.where(m2, l1vr, V0)
    R0n, V0n = plsc.sort_key_val(ak, av)
    R1n, V1n = plsc.sort_key_val(bk, bv)
    return R0n, V0n, R1n, V1n


def _sc_body(px_h, py_h, pz_h, ox_h, oy_h, oz_h, pm_h,
             dist_h, nidx_h, valid_h,
             pxv, pyv, pzv, oxv, oyv, ozv, prm,
             shx, shy, shz, fidx, fqx, fqy, fqz,
             vkey, vidx, ikey, iidx,
             od, oi, ov, csel):
    b = lax.axis_index("c") * 16 + lax.axis_index("s")

    pltpu.sync_copy(px_h.at[b], pxv.at[pl.ds(0, _N)])
    pltpu.sync_copy(py_h.at[b], pyv.at[pl.ds(0, _N)])
    pltpu.sync_copy(pz_h.at[b], pzv.at[pl.ds(0, _N)])
    pltpu.sync_copy(ox_h.at[b], oxv.at[pl.ds(0, _N)])
    pltpu.sync_copy(oy_h.at[b], oyv.at[pl.ds(0, _N)])
    pltpu.sync_copy(oz_h.at[b], ozv.at[pl.ds(0, _N)])
    pltpu.sync_copy(pm_h.at[b], prm.at[pl.ds(0, _N)])

    lane = lax.iota(jnp.int32, 16)
    lane27 = lane * _NCELL

    def pre_body(t, _):
        c = t // 8
        jv16 = (t % 8) * 16
        sl = t * 16
        oxs = oxv[pl.ds(c, 16)][0]
        oys = oyv[pl.ds(c, 16)][0]
        ozs = ozv[pl.ds(c, 16)][0]
        shx[pl.ds(sl, 16)] = pxv[pl.ds(jv16, 16)] + oxs
        shy[pl.ds(sl, 16)] = pyv[pl.ds(jv16, 16)] + oys
        shz[pl.ds(sl, 16)] = pzv[pl.ds(jv16, 16)] + ozs
        fidx[pl.ds(sl, 16)] = lane27 + (jv16 * _NCELL + c)
        return 0

    lax.fori_loop(0, _NV, pre_body, 0)

    civ = prm[pl.ds(0, 16)]
    c00, c10, c20 = civ[0], civ[3], civ[6]
    c01, c11, c21 = civ[1], civ[4], civ[7]
    c02, c12, c22 = civ[2], civ[5], civ[8]
    hx, hy, hz = civ[9], civ[10], civ[11]

    def fq_body(jv, _):
        sl = jv * 16
        bx = pxv[pl.ds(sl, 16)]
        by = pyv[pl.ds(sl, 16)]
        bz = pzv[pl.ds(sl, 16)]
        fqx[pl.ds(sl, 16)] = bx * c00 + by * c10 + bz * c20
        fqy[pl.ds(sl, 16)] = bx * c01 + by * c11 + bz * c21
        fqz[pl.ds(sl, 16)] = bx * c02 + by * c12 + bz * c22
        return 0

    lax.fori_loop(0, 8, fq_body, 0)

    def query_body(i, _):
        qx = pxv[pl.ds(i, 16)][0]
        qy = pyv[pl.ds(i, 16)][0]
        qz = pzv[pl.ds(i, 16)][0]
        fx = fqx[pl.ds(i, 16)][0]
        fy = fqy[pl.ds(i, 16)][0]
        fz = fqz[pl.ds(i, 16)][0]

        rpad = _CUTOFF2 ** 0.5 + 0.01
        bn = (jnp.logical_not(fx * hx > rpad),
              jnp.logical_not(fy * hy > rpad),
              jnp.logical_not(fz * hz > rpad))
        bp = (jnp.logical_not((1.0 - fx) * hx > rpad),
              jnp.logical_not((1.0 - fy) * hy > rpad),
              jnp.logical_not((1.0 - fz) * hz > rpad))
        cnt = jnp.int32(0)
        for c in range(_NCELL):
            ks = (c // 9 - 1, (c // 3) % 3 - 1, c % 3 - 1)
            keep = None
            for d in range(3):
                cond = bn[d] if ks[d] == -1 else (bp[d] if ks[d] == 1 else None)
                if cond is not None:
                    keep = cond if keep is None else jnp.logical_and(keep, cond)
            csel[cnt] = jnp.int32(c)
            cnt = cnt + (jnp.int32(1) if keep is None
                         else keep.astype(jnp.int32))

        @plsc.parallel_loop(0, cnt * 8, unroll=16,
                            carry=jnp.zeros((16,), jnp.int32))
        def pass1(t, off_spl):
            c = csel[t >> 3]
            jv = t & 7
            sl = c * _N + jv * 16
            dx = qx - shx[pl.ds(sl, 16)]
            dy = qy - shy[pl.ds(sl, 16)]
            dz = qz - shz[pl.ds(sl, 16)]
            d2 = dx * dx + dy * dy + dz * dz
            ok = (d2 > 1e-4) & (d2 <= _CUTOFF2)
            oki = ok.astype(jnp.int32)
            inc = plsc.cumsum(oki)
            posn = off_spl + inc - oki
            fl = lane27 + (jv * 16 * _NCELL + c)
            plsc.store_scatter(vkey, [posn], d2, mask=ok)
            plsc.store_scatter(vidx, [posn], fl, mask=ok)
            return off_spl + plsc.all_reduce_population_count(ok)

        mv = pass1[0]
        vkey[pl.ds(mv, 16)] = jnp.full((16,), _PAD_KEY, jnp.float32)
        vidx[pl.ds(mv, 16)] = jnp.zeros((16,), jnp.int32)

        R0 = jnp.full((16,), _PAD_KEY, jnp.float32)
        R1 = jnp.full((16,), _PAD_KEY, jnp.float32)
        V0 = jnp.zeros((16,), jnp.int32)
        V1 = jnp.zeros((16,), jnp.int32)

        def mbody(t, carry):
            R0, V0, R1, V1 = carry
            sl = t * 16
            return _merge32(R0, V0, R1, V1, vkey[pl.ds(sl, 16)],
                            vidx[pl.ds(sl, 16)])

        nvv = (mv + 15) // 16
        R0, V0, R1, V1 = lax.fori_loop(0, nvv, mbody, (R0, V0, R1, V1))

        def pass2(t, ioff):
            sl = t * 16
            dx = qx - shx[pl.ds(sl, 16)]
            dy = qy - shy[pl.ds(sl, 16)]
            dz = qz - shz[pl.ds(sl, 16)]
            d2 = dx * dx + dy * dy + dz * dz
            bad = (d2 <= 1e-4) | (d2 > _CUTOFF2)
            fl = fidx[pl.ds(sl, 16)]
            fkey = _INVALID_BASE + fl.astype(jnp.float32)
            plsc.store_compressed(ikey.at[pl.ds(ioff, 16)], fkey, mask=bad)
            plsc.store_compressed(iidx.at[pl.ds(ioff, 16)], fl, mask=bad)
            return ioff + jnp.sum(bad.astype(jnp.int32))

        t2 = jnp.where(mv < _TOPK, _NV, 0)
        ioff = lax.fori_loop(0, t2, pass2, jnp.int32(0))
        ikey[pl.ds(ioff, 16)] = jnp.full((16,), _PAD_KEY, jnp.float32)
        iidx[pl.ds(ioff, 16)] = jnp.zeros((16,), jnp.int32)

        def mbody2(t, carry):
            R0, V0, R1, V1 = carry
            sl = t * 16
            return _merge32(R0, V0, R1, V1, ikey[pl.ds(sl, 16)],
                            iidx[pl.ds(sl, 16)])

        nvi = jnp.where(mv < _TOPK, (ioff + 15) // 16, 0)
        R0, V0, R1, V1 = lax.fori_loop(0, nvi, mbody2, (R0, V0, R1, V1))

        base = i * _TOPK
        for k0, (rk, rv) in ((0, (R0, V0)), (16, (R1, V1))):
            sel = rk < _INVALID_BASE
            dist = jnp.where(sel, _sqrt16(rk), 0.0)
            od[pl.ds(base + k0, 16)] = dist
            oi[pl.ds(base + k0, 16)] = rv // _NCELL
            ov[pl.ds(base + k0, 16)] = sel.astype(jnp.int32)
        return 0

    lax.fori_loop(0, _N, query_body, 0)

    pltpu.sync_copy(od, dist_h.at[b])
    pltpu.sync_copy(oi, nidx_h.at[b])
    pltpu.sync_copy(ov, valid_h.at[b])


@jax.jit
def kernel(pos, cell):
    B, n, _ = pos.shape
    r = jnp.arange(-1, 2, dtype=pos.dtype)
    gx, gy, gz = jnp.meshgrid(r, r, r, indexing='ij')
    offsets = jnp.stack([gx.ravel(), gy.ravel(), gz.ravel()], axis=-1)
    cart_off = jnp.einsum('cd,bde->bce', offsets, cell)
    cart_off = jnp.pad(cart_off, ((0, 0), (0, n - _NCELL), (0, 0)))

    px, py, pz = pos[..., 0], pos[..., 1], pos[..., 2]
    ox, oy, oz = cart_off[..., 0], cart_off[..., 1], cart_off[..., 2]

    cinv = jnp.linalg.inv(cell)
    hspc = 1.0 / jnp.sqrt(jnp.sum(cinv * cinv, axis=1))
    params = jnp.concatenate([cinv.reshape(B, 9), hspc], axis=1)
    params = jnp.pad(params, ((0, 0), (0, n - 12)))

    mesh = plsc.VectorSubcoreMesh(core_axis_name="c", subcore_axis_name="s")
    out_type = [
        jax.ShapeDtypeStruct((B, n * _TOPK), jnp.float32),
        jax.ShapeDtypeStruct((B, n * _TOPK), jnp.int32),
        jax.ShapeDtypeStruct((B, n * _TOPK), jnp.int32),
    ]
    scratch = [
        pltpu.VMEM((n + 16,), jnp.float32),
        pltpu.VMEM((n + 16,), jnp.float32),
        pltpu.VMEM((n + 16,), jnp.float32),
        pltpu.VMEM((n + 16,), jnp.float32),
        pltpu.VMEM((n + 16,), jnp.float32),
        pltpu.VMEM((n + 16,), jnp.float32),
        pltpu.VMEM((n + 16,), jnp.float32),
        pltpu.VMEM((_CAND,), jnp.float32),
        pltpu.VMEM((_CAND,), jnp.float32),
        pltpu.VMEM((_CAND,), jnp.float32),
        pltpu.VMEM((_CAND,), jnp.int32),
        pltpu.VMEM((n + 16,), jnp.float32),
        pltpu.VMEM((n + 16,), jnp.float32),
        pltpu.VMEM((n + 16,), jnp.float32),
        pltpu.VMEM((_CAND + 16,), jnp.float32),
        pltpu.VMEM((_CAND + 16,), jnp.int32),
        pltpu.VMEM((_CAND + 16,), jnp.float32),
        pltpu.VMEM((_CAND + 16,), jnp.int32),
        pltpu.VMEM((n * _TOPK,), jnp.float32),
        pltpu.VMEM((n * _TOPK,), jnp.int32),
        pltpu.VMEM((n * _TOPK,), jnp.int32),
        pltpu.SMEM((32,), jnp.int32),
    ]
    fn = pl.kernel(_sc_body, mesh=mesh, out_type=out_type,
                   compiler_params=pltpu.CompilerParams(
                       needs_layout_passes=False),
                   scratch_types=scratch)
    dist_f, nidx_f, valid_f = fn(px, py, pz, ox, oy, oz, params)

    dist = dist_f.reshape(B, n, _TOPK)
    nidx = nidx_f.reshape(B, n, _TOPK)
    valid = valid_f.reshape(B, n, _TOPK).astype(bool)
    return dist, nidx, valid


# speedup vs baseline: 1.3755x; 1.3755x over previous
import functools

import jax
import jax.numpy as jnp
from jax import lax
from jax.experimental import pallas as pl
from jax.experimental.pallas import tpu as pltpu
from jax.experimental.pallas import tpu_sc as plsc

_N = 128
_NCELL = 27
_TOPK = 32
_CUTOFF2 = 25.0
_INVALID_BASE = 100000.0
_PAD_KEY = 1e9
_NV = (_N * _NCELL) // 16
_CAND = _N * _NCELL


def _sqrt16(x):
    xi = lax.bitcast_convert_type(x, jnp.int32)
    yi = (xi >> 1) + jnp.int32(0x1FBD1DF5)
    y = lax.bitcast_convert_type(yi, jnp.float32)
    for _ in range(2):
        y = 0.5 * (y + x / y)
    return y


def _merge32(R0, V0, R1, V1, ck, cv):
    cs, cvs = plsc.sort_key_val(ck, cv)
    cr = lax.rev(cs, (0,))
    cvr = lax.rev(cvs, (0,))
    m1 = R1 <= cr
    lo_k = jnp.where(m1, R1, cr)
    lo_v = jnp.where(m1, V1, cvr)
    l1k, l1v = plsc.sort_key_val(lo_k, lo_v)
    l1kr = lax.rev(l1k, (0,))
    l1vr = lax.rev(l1v, (0,))
    m2 = R0 <= l1kr
    ak = jnp.where(m2, R0, l1kr)
    av = jnp.where(m2, V0, l1vr)
    bk = jnp.where(m2, l1kr, R0)
    bv = jnp.where(m2, l1vr, V0)
    R0n, V0n = plsc.sort_key_val(ak, av)
    R1n, V1n = plsc.sort_key_val(bk, bv)
    return R0n, V0n, R1n, V1n


def _sc_body(px_h, py_h, pz_h, ox_h, oy_h, oz_h, pm_h,
             dist_h, nidx_h, valid_h,
             pxv, pyv, pzv, oxv, oyv, ozv, prm,
             shx, shy, shz, fidx, fqx, fqy, fqz,
             vkey, vidx, ikey, iidx,
             od, oi, ov, csel):
    b = lax.axis_index("c") * 16 + lax.axis_index("s")

    pltpu.sync_copy(px_h.at[b], pxv.at[pl.ds(0, _N)])
    pltpu.sync_copy(py_h.at[b], pyv.at[pl.ds(0, _N)])
    pltpu.sync_copy(pz_h.at[b], pzv.at[pl.ds(0, _N)])
    pltpu.sync_copy(ox_h.at[b], oxv.at[pl.ds(0, _N)])
    pltpu.sync_copy(oy_h.at[b], oyv.at[pl.ds(0, _N)])
    pltpu.sync_copy(oz_h.at[b], ozv.at[pl.ds(0, _N)])
    pltpu.sync_copy(pm_h.at[b], prm.at[pl.ds(0, _N)])

    lane = lax.iota(jnp.int32, 16)
    lane27 = lane * _NCELL

    def pre_body(t, _):
        c = t // 8
        jv16 = (t % 8) * 16
        sl = t * 16
        oxs = oxv[pl.ds(c, 16)][0]
        oys = oyv[pl.ds(c, 16)][0]
        ozs = ozv[pl.ds(c, 16)][0]
        shx[pl.ds(sl, 16)] = pxv[pl.ds(jv16, 16)] + oxs
        shy[pl.ds(sl, 16)] = pyv[pl.ds(jv16, 16)] + oys
        shz[pl.ds(sl, 16)] = pzv[pl.ds(jv16, 16)] + ozs
        fidx[pl.ds(sl, 16)] = lane27 + (jv16 * _NCELL + c)
        return 0

    lax.fori_loop(0, _NV, pre_body, 0)

    civ = prm[pl.ds(0, 16)]
    c00, c10, c20 = civ[0], civ[3], civ[6]
    c01, c11, c21 = civ[1], civ[4], civ[7]
    c02, c12, c22 = civ[2], civ[5], civ[8]
    hx, hy, hz = civ[9], civ[10], civ[11]

    def fq_body(jv, _):
        sl = jv * 16
        bx = pxv[pl.ds(sl, 16)]
        by = pyv[pl.ds(sl, 16)]
        bz = pzv[pl.ds(sl, 16)]
        fqx[pl.ds(sl, 16)] = bx * c00 + by * c10 + bz * c20
        fqy[pl.ds(sl, 16)] = bx * c01 + by * c11 + bz * c21
        fqz[pl.ds(sl, 16)] = bx * c02 + by * c12 + bz * c22
        return 0

    lax.fori_loop(0, 8, fq_body, 0)

    def query_body(i, _):
        qx = pxv[pl.ds(i, 16)][0]
        qy = pyv[pl.ds(i, 16)][0]
        qz = pzv[pl.ds(i, 16)][0]
        fx = fqx[pl.ds(i, 16)][0]
        fy = fqy[pl.ds(i, 16)][0]
        fz = fqz[pl.ds(i, 16)][0]

        rpad = _CUTOFF2 ** 0.5 + 0.01
        bn = (jnp.logical_not(fx * hx > rpad),
              jnp.logical_not(fy * hy > rpad),
              jnp.logical_not(fz * hz > rpad))
        bp = (jnp.logical_not((1.0 - fx) * hx > rpad),
              jnp.logical_not((1.0 - fy) * hy > rpad),
              jnp.logical_not((1.0 - fz) * hz > rpad))
        cnt = jnp.int32(0)
        for c in range(_NCELL):
            ks = (c // 9 - 1, (c // 3) % 3 - 1, c % 3 - 1)
            keep = None
            for d in range(3):
                cond = bn[d] if ks[d] == -1 else (bp[d] if ks[d] == 1 else None)
                if cond is not None:
                    keep = cond if keep is None else jnp.logical_and(keep, cond)
            csel[cnt] = jnp.int32(c)
            cnt = cnt + (jnp.int32(1) if keep is None
                         else keep.astype(jnp.int32))

        @plsc.parallel_loop(0, cnt * 8, unroll=4,
                            carry=jnp.zeros((16,), jnp.int32))
        def pass1(t, off_spl):
            c = csel[t >> 3]
            jv = t & 7
            sl = c * _N + jv * 16
            dx = qx - shx[pl.ds(sl, 16)]
            dy = qy - shy[pl.ds(sl, 16)]
            dz = qz - shz[pl.ds(sl, 16)]
            d2 = dx * dx + dy * dy + dz * dz
            ok = (d2 > 1e-4) & (d2 <= _CUTOFF2)
            oki = ok.astype(jnp.int32)
            inc = plsc.cumsum(oki)
            posn = off_spl + inc - oki
            fl = lane27 + (jv * 16 * _NCELL + c)
            plsc.store_scatter(vkey, [posn], d2, mask=ok)
            plsc.store_scatter(vidx, [posn], fl, mask=ok)
            return off_spl + plsc.all_reduce_population_count(ok)

        mv = pass1[0]
        vkey[pl.ds(mv, 16)] = jnp.full((16,), _PAD_KEY, jnp.float32)
        vidx[pl.ds(mv, 16)] = jnp.zeros((16,), jnp.int32)

        R0 = jnp.full((16,), _PAD_KEY, jnp.float32)
        R1 = jnp.full((16,), _PAD_KEY, jnp.float32)
        V0 = jnp.zeros((16,), jnp.int32)
        V1 = jnp.zeros((16,), jnp.int32)

        def mbody(t, carry):
            R0, V0, R1, V1 = carry
            sl = t * 16
            return _merge32(R0, V0, R1, V1, vkey[pl.ds(sl, 16)],
                            vidx[pl.ds(sl, 16)])

        nvv = (mv + 15) // 16
        R0, V0, R1, V1 = lax.fori_loop(0, nvv, mbody, (R0, V0, R1, V1))

        def pass2(t, ioff):
            sl = t * 16
            dx = qx - shx[pl.ds(sl, 16)]
            dy = qy - shy[pl.ds(sl, 16)]
            dz = qz - shz[pl.ds(sl, 16)]
            d2 = dx * dx + dy * dy + dz * dz
            bad = (d2 <= 1e-4) | (d2 > _CUTOFF2)
            fl = fidx[pl.ds(sl, 16)]
            fkey = _INVALID_BASE + fl.astype(jnp.float32)
            plsc.store_compressed(ikey.at[pl.ds(ioff, 16)], fkey, mask=bad)
            plsc.store_compressed(iidx.at[pl.ds(ioff, 16)], fl, mask=bad)
            return ioff + jnp.sum(bad.astype(jnp.int32))

        t2 = jnp.where(mv < _TOPK, _NV, 0)
        ioff = lax.fori_loop(0, t2, pass2, jnp.int32(0))
        ikey[pl.ds(ioff, 16)] = jnp.full((16,), _PAD_KEY, jnp.float32)
        iidx[pl.ds(ioff, 16)] = jnp.zeros((16,), jnp.int32)

        def mbody2(t, carry):
            R0, V0, R1, V1 = carry
            sl = t * 16
            return _merge32(R0, V0, R1, V1, ikey[pl.ds(sl, 16)],
                            iidx[pl.ds(sl, 16)])

        nvi = jnp.where(mv < _TOPK, (ioff + 15) // 16, 0)
        R0, V0, R1, V1 = lax.fori_loop(0, nvi, mbody2, (R0, V0, R1, V1))

        base = i * _TOPK
        for k0, (rk, rv) in ((0, (R0, V0)), (16, (R1, V1))):
            sel = rk < _INVALID_BASE
            dist = jnp.where(sel, _sqrt16(rk), 0.0)
            od[pl.ds(base + k0, 16)] = dist
            oi[pl.ds(base + k0, 16)] = rv // _NCELL
            ov[pl.ds(base + k0, 16)] = sel.astype(jnp.int32)
        return 0

    lax.fori_loop(0, _N, query_body, 0)

    pltpu.sync_copy(od, dist_h.at[b])
    pltpu.sync_copy(oi, nidx_h.at[b])
    pltpu.sync_copy(ov, valid_h.at[b])


@jax.jit
def kernel(pos, cell):
    B, n, _ = pos.shape
    r = jnp.arange(-1, 2, dtype=pos.dtype)
    gx, gy, gz = jnp.meshgrid(r, r, r, indexing='ij')
    offsets = jnp.stack([gx.ravel(), gy.ravel(), gz.ravel()], axis=-1)
    cart_off = jnp.einsum('cd,bde->bce', offsets, cell)
    cart_off = jnp.pad(cart_off, ((0, 0), (0, n - _NCELL), (0, 0)))

    px, py, pz = pos[..., 0], pos[..., 1], pos[..., 2]
    ox, oy, oz = cart_off[..., 0], cart_off[..., 1], cart_off[..., 2]

    cinv = jnp.linalg.inv(cell)
    hspc = 1.0 / jnp.sqrt(jnp.sum(cinv * cinv, axis=1))
    params = jnp.concatenate([cinv.reshape(B, 9), hspc], axis=1)
    params = jnp.pad(params, ((0, 0), (0, n - 12)))

    mesh = plsc.VectorSubcoreMesh(core_axis_name="c", subcore_axis_name="s")
    out_type = [
        jax.ShapeDtypeStruct((B, n * _TOPK), jnp.float32),
        jax.ShapeDtypeStruct((B, n * _TOPK), jnp.int32),
        jax.ShapeDtypeStruct((B, n * _TOPK), jnp.int32),
    ]
    scratch = [
        pltpu.VMEM((n + 16,), jnp.float32),
        pltpu.VMEM((n + 16,), jnp.float32),
        pltpu.VMEM((n + 16,), jnp.float32),
        pltpu.VMEM((n + 16,), jnp.float32),
        pltpu.VMEM((n + 16,), jnp.float32),
        pltpu.VMEM((n + 16,), jnp.float32),
        pltpu.VMEM((n + 16,), jnp.float32),
        pltpu.VMEM((_CAND,), jnp.float32),
        pltpu.VMEM((_CAND,), jnp.float32),
        pltpu.VMEM((_CAND,), jnp.float32),
        pltpu.VMEM((_CAND,), jnp.int32),
        pltpu.VMEM((n + 16,), jnp.float32),
        pltpu.VMEM((n + 16,), jnp.float32),
        pltpu.VMEM((n + 16,), jnp.float32),
        pltpu.VMEM((_CAND + 16,), jnp.float32),
        pltpu.VMEM((_CAND + 16,), jnp.int32),
        pltpu.VMEM((_CAND + 16,), jnp.float32),
        pltpu.VMEM((_CAND + 16,), jnp.int32),
        pltpu.VMEM((n * _TOPK,), jnp.float32),
        pltpu.VMEM((n * _TOPK,), jnp.int32),
        pltpu.VMEM((n * _TOPK,), jnp.int32),
        pltpu.SMEM((32,), jnp.int32),
    ]
    fn = pl.kernel(_sc_body, mesh=mesh, out_type=out_type,
                   compiler_params=pltpu.CompilerParams(
                       needs_layout_passes=False),
                   scratch_types=scratch)
    dist_f, nidx_f, valid_f = fn(px, py, pz, ox, oy, oz, params)

    dist = dist_f.reshape(B, n, _TOPK)
    nidx = nidx_f.reshape(B, n, _TOPK)
    valid = valid_f.reshape(B, n, _TOPK).astype(bool)
    return dist, nidx, valid


# speedup vs baseline: 1.3759x; 1.0003x over previous
import jax
import jax.numpy as jnp
from jax import lax
from jax.experimental import pallas as pl
from jax.experimental.pallas import tpu as pltpu
from jax.experimental.pallas import tpu_sc as plsc

_N = 128
_NCELL = 27
_TOPK = 32
_CUTOFF2 = 25.0
_INVALID_BASE = 100000.0
_PAD_KEY = 1e9
_NV = (_N * _NCELL) // 16
_CAND = _N * _NCELL


def _sqrt16(x):
    xi = lax.bitcast_convert_type(x, jnp.int32)
    yi = (xi >> 1) + jnp.int32(0x1FBD1DF5)
    y = lax.bitcast_convert_type(yi, jnp.float32)
    for _ in range(2):
        y = 0.5 * (y + x / y)
    return y


def _merge32(R0, V0, R1, V1, ck, cv):
    cs, cvs = plsc.sort_key_val(ck, cv)
    cr = lax.rev(cs, (0,))
    cvr = lax.rev(cvs, (0,))
    m1 = R1 <= cr
    lo_k = jnp.where(m1, R1, cr)
    lo_v = jnp.where(m1, V1, cvr)
    l1k, l1v = plsc.sort_key_val(lo_k, lo_v)
    l1kr = lax.rev(l1k, (0,))
    l1vr = lax.rev(l1v, (0,))
    m2 = R0 <= l1kr
    ak = jnp.where(m2, R0, l1kr)
    av = jnp.where(m2, V0, l1vr)
    bk = jnp.where(m2, l1kr, R0)
    bv = jnp.where(m2, l1vr, V0)
    R0n, V0n = plsc.sort_key_val(ak, av)
    R1n, V1n = plsc.sort_key_val(bk, bv)
    return R0n, V0n, R1n, V1n


def _sc_body(px_h, py_h, pz_h, ox_h, oy_h, oz_h, pm_h,
             dist_h, nidx_h, valid_h,
             pxv, pyv, pzv, oxv, oyv, ozv, prm,
             shx, shy, shz, fidx, fqx, fqy, fqz,
             vkey, vidx, ikey, iidx,
             od, oi, ov, csel):
    b = lax.axis_index("c") * 16 + lax.axis_index("s")

    pltpu.sync_copy(px_h.at[b], pxv.at[pl.ds(0, _N)])
    pltpu.sync_copy(py_h.at[b], pyv.at[pl.ds(0, _N)])
    pltpu.sync_copy(pz_h.at[b], pzv.at[pl.ds(0, _N)])
    pltpu.sync_copy(ox_h.at[b], oxv.at[pl.ds(0, _N)])
    pltpu.sync_copy(oy_h.at[b], oyv.at[pl.ds(0, _N)])
    pltpu.sync_copy(oz_h.at[b], ozv.at[pl.ds(0, _N)])
    pltpu.sync_copy(pm_h.at[b], prm.at[pl.ds(0, _N)])

    lane = lax.iota(jnp.int32, 16)
    lane27 = lane * _NCELL

    def pre_body(t, _):
        c = t // 8
        jv16 = (t % 8) * 16
        sl = t * 16
        oxs = oxv[pl.ds(c, 16)][0]
        oys = oyv[pl.ds(c, 16)][0]
        ozs = ozv[pl.ds(c, 16)][0]
        shx[pl.ds(sl, 16)] = pxv[pl.ds(jv16, 16)] + oxs
        shy[pl.ds(sl, 16)] = pyv[pl.ds(jv16, 16)] + oys
        shz[pl.ds(sl, 16)] = pzv[pl.ds(jv16, 16)] + ozs
        fidx[pl.ds(sl, 16)] = lane27 + (jv16 * _NCELL + c)
        return 0

    lax.fori_loop(0, _NV, pre_body, 0)

    civ = prm[pl.ds(0, 16)]
    c00, c10, c20 = civ[0], civ[3], civ[6]
    c01, c11, c21 = civ[1], civ[4], civ[7]
    c02, c12, c22 = civ[2], civ[5], civ[8]
    hx, hy, hz = civ[9], civ[10], civ[11]

    def fq_body(jv, _):
        sl = jv * 16
        bx = pxv[pl.ds(sl, 16)]
        by = pyv[pl.ds(sl, 16)]
        bz = pzv[pl.ds(sl, 16)]
        fqx[pl.ds(sl, 16)] = bx * c00 + by * c10 + bz * c20
        fqy[pl.ds(sl, 16)] = bx * c01 + by * c11 + bz * c21
        fqz[pl.ds(sl, 16)] = bx * c02 + by * c12 + bz * c22
        return 0

    lax.fori_loop(0, 8, fq_body, 0)

    def query_body(i, _):
        qx = pxv[pl.ds(i, 16)][0]
        qy = pyv[pl.ds(i, 16)][0]
        qz = pzv[pl.ds(i, 16)][0]
        fx = fqx[pl.ds(i, 16)][0]
        fy = fqy[pl.ds(i, 16)][0]
        fz = fqz[pl.ds(i, 16)][0]

        rpad = _CUTOFF2 ** 0.5 + 0.01
        bn = (jnp.logical_not(fx * hx > rpad),
              jnp.logical_not(fy * hy > rpad),
              jnp.logical_not(fz * hz > rpad))
        bp = (jnp.logical_not((1.0 - fx) * hx > rpad),
              jnp.logical_not((1.0 - fy) * hy > rpad),
              jnp.logical_not((1.0 - fz) * hz > rpad))
        cnt = jnp.int32(0)
        for c in range(_NCELL):
            ks = (c // 9 - 1, (c // 3) % 3 - 1, c % 3 - 1)
            keep = None
            for d in range(3):
                cond = bn[d] if ks[d] == -1 else (bp[d] if ks[d] == 1 else None)
                if cond is not None:
                    keep = cond if keep is None else jnp.logical_and(keep, cond)
            csel[cnt] = jnp.int32(c)
            cnt = cnt + (jnp.int32(1) if keep is None
                         else keep.astype(jnp.int32))

        @plsc.parallel_loop(0, cnt * 8, unroll=4,
                            carry=jnp.zeros((16,), jnp.int32))
        def pass1(t, off_spl):
            c = csel[t >> 3]
            jv = t & 7
            sl = c * _N + jv * 16
            dx = qx - shx[pl.ds(sl, 16)]
            dy = qy - shy[pl.ds(sl, 16)]
            dz = qz - shz[pl.ds(sl, 16)]
            d2 = dx * dx + dy * dy + dz * dz
            ok = (d2 > 1e-4) & (d2 <= _CUTOFF2)
            oki = ok.astype(jnp.int32)
            inc = plsc.cumsum(oki)
            posn = off_spl + inc - oki
            fl = lane27 + (jv * 16 * _NCELL + c)
            plsc.store_scatter(vkey, [posn], d2, mask=ok)
            plsc.store_scatter(vidx, [posn], fl, mask=ok)
            return off_spl + plsc.all_reduce_population_count(ok)

        mv = pass1[0]
        vkey[pl.ds(mv, 16)] = jnp.full((16,), _PAD_KEY, jnp.float32)
        vidx[pl.ds(mv, 16)] = jnp.zeros((16,), jnp.int32)

        R0 = jnp.full((16,), _PAD_KEY, jnp.float32)
        R1 = jnp.full((16,), _PAD_KEY, jnp.float32)
        V0 = jnp.zeros((16,), jnp.int32)
        V1 = jnp.zeros((16,), jnp.int32)

        def mbody(t, carry):
            R0, V0, R1, V1 = carry
            sl = t * 16
            return _merge32(R0, V0, R1, V1, vkey[pl.ds(sl, 16)],
                            vidx[pl.ds(sl, 16)])

        nvv = (mv + 15) // 16
        R0, V0, R1, V1 = lax.fori_loop(0, nvv, mbody, (R0, V0, R1, V1))

        def pass2(t, ioff):
            sl = t * 16
            dx = qx - shx[pl.ds(sl, 16)]
            dy = qy - shy[pl.ds(sl, 16)]
            dz = qz - shz[pl.ds(sl, 16)]
            d2 = dx * dx + dy * dy + dz * dz
            bad = (d2 <= 1e-4) | (d2 > _CUTOFF2)
            fl = fidx[pl.ds(sl, 16)]
            fkey = _INVALID_BASE + fl.astype(jnp.float32)
            plsc.store_compressed(ikey.at[pl.ds(ioff, 16)], fkey, mask=bad)
            plsc.store_compressed(iidx.at[pl.ds(ioff, 16)], fl, mask=bad)
            return ioff + jnp.sum(bad.astype(jnp.int32))

        t2 = jnp.where(mv < _TOPK, _NV, 0)
        ioff = lax.fori_loop(0, t2, pass2, jnp.int32(0))
        ikey[pl.ds(ioff, 16)] = jnp.full((16,), _PAD_KEY, jnp.float32)
        iidx[pl.ds(ioff, 16)] = jnp.zeros((16,), jnp.int32)

        def mbody2(t, carry):
            R0, V0, R1, V1 = carry
            sl = t * 16
            return _merge32(R0, V0, R1, V1, ikey[pl.ds(sl, 16)],
                            iidx[pl.ds(sl, 16)])

        nvi = jnp.where(mv < _TOPK, (ioff + 15) // 16, 0)
        R0, V0, R1, V1 = lax.fori_loop(0, nvi, mbody2, (R0, V0, R1, V1))

        base = i * _TOPK
        for k0, (rk, rv) in ((0, (R0, V0)), (16, (R1, V1))):
            sel = rk < _INVALID_BASE
            dist = jnp.where(sel, _sqrt16(rk), 0.0)
            od[pl.ds(base + k0, 16)] = dist
            oi[pl.ds(base + k0, 16)] = rv // _NCELL
            ov[pl.ds(base + k0, 16)] = sel.astype(jnp.int32)
        return 0

    lax.fori_loop(0, _N, query_body, 0)

    pltpu.sync_copy(od, dist_h.at[b])
    pltpu.sync_copy(oi, nidx_h.at[b])
    pltpu.sync_copy(ov, valid_h.at[b])


@jax.jit
def kernel(pos, cell):
    B, n, _ = pos.shape
    r = jnp.arange(-1, 2, dtype=pos.dtype)
    gx, gy, gz = jnp.meshgrid(r, r, r, indexing='ij')
    offsets = jnp.stack([gx.ravel(), gy.ravel(), gz.ravel()], axis=-1)
    cart_off = jnp.einsum('cd,bde->bce', offsets, cell)
    cart_off = jnp.pad(cart_off, ((0, 0), (0, n - _NCELL), (0, 0)))

    px, py, pz = pos[..., 0], pos[..., 1], pos[..., 2]
    ox, oy, oz = cart_off[..., 0], cart_off[..., 1], cart_off[..., 2]

    cinv = jnp.linalg.inv(cell)
    hspc = 1.0 / jnp.sqrt(jnp.sum(cinv * cinv, axis=1))
    params = jnp.concatenate([cinv.reshape(B, 9), hspc], axis=1)
    params = jnp.pad(params, ((0, 0), (0, n - 12)))

    mesh = plsc.VectorSubcoreMesh(core_axis_name="c", subcore_axis_name="s")
    out_type = [
        jax.ShapeDtypeStruct((B, n * _TOPK), jnp.float32),
        jax.ShapeDtypeStruct((B, n * _TOPK), jnp.int32),
        jax.ShapeDtypeStruct((B, n * _TOPK), jnp.int32),
    ]
    scratch = [
        pltpu.VMEM((n + 16,), jnp.float32),
        pltpu.VMEM((n + 16,), jnp.float32),
        pltpu.VMEM((n + 16,), jnp.float32),
        pltpu.VMEM((n + 16,), jnp.float32),
        pltpu.VMEM((n + 16,), jnp.float32),
        pltpu.VMEM((n + 16,), jnp.float32),
        pltpu.VMEM((n + 16,), jnp.float32),
        pltpu.VMEM((_CAND,), jnp.float32),
        pltpu.VMEM((_CAND,), jnp.float32),
        pltpu.VMEM((_CAND,), jnp.float32),
        pltpu.VMEM((_CAND,), jnp.int32),
        pltpu.VMEM((n + 16,), jnp.float32),
        pltpu.VMEM((n + 16,), jnp.float32),
        pltpu.VMEM((n + 16,), jnp.float32),
        pltpu.VMEM((_CAND + 16,), jnp.float32),
        pltpu.VMEM((_CAND + 16,), jnp.int32),
        pltpu.VMEM((_CAND + 16,), jnp.float32),
        pltpu.VMEM((_CAND + 16,), jnp.int32),
        pltpu.VMEM((n * _TOPK,), jnp.float32),
        pltpu.VMEM((n * _TOPK,), jnp.int32),
        pltpu.VMEM((n * _TOPK,), jnp.int32),
        pltpu.SMEM((32,), jnp.int32),
    ]
    fn = pl.kernel(_sc_body, mesh=mesh, out_type=out_type,
                   compiler_params=pltpu.CompilerParams(
                       needs_layout_passes=False),
                   scratch_types=scratch)
    dist_f, nidx_f, valid_f = fn(px, py, pz, ox, oy, oz, params)

    dist = dist_f.reshape(B, n, _TOPK)
    nidx = nidx_f.reshape(B, n, _TOPK)
    valid = valid_f.reshape(B, n, _TOPK).astype(bool)
    return dist, nidx, valid
